# baseline (speedup 1.0000x reference)
import functools

import jax
import jax.numpy as jnp
from jax import lax
from jax.experimental import pallas as pl
from jax.experimental.pallas import tpu as pltpu
from jax.experimental.pallas import tpu_sc as plsc

E = 64
D = 1024
F = 1024
N = 2048
TM = 128
NT = N // TM
G = NT + E
RB = 256
NRB = N // RB


def _routing_body(x_ref, wg_ref, bg_ref, ti_ref, rank_ref, offs_ref, meta_ref,
                  run_ref):
    i = pl.program_id(0)

    @pl.when(i == 0)
    def _init():
        run_ref[...] = jnp.zeros_like(run_ref)

    logits = jnp.dot(x_ref[...], wg_ref[...],
                     preferred_element_type=jnp.float32) + bg_ref[...]
    ti = jnp.argmax(logits, axis=-1).astype(jnp.int32)
    onehot = (ti[:, None] == lax.broadcasted_iota(jnp.int32, (1, E), 1)
              ).astype(jnp.float32)
    r = lax.broadcasted_iota(jnp.int32, (RB, RB), 0)
    c = lax.broadcasted_iota(jnp.int32, (RB, RB), 1)
    ltri = (c < r).astype(jnp.float32)
    prefix = jnp.dot(ltri, onehot, preferred_element_type=jnp.float32)
    totals = run_ref[...]
    rank = jnp.sum((prefix + totals) * onehot, axis=-1)
    ti_ref[...] = ti
    rank_ref[...] = rank.astype(jnp.int32)
    run_ref[...] = totals + jnp.sum(onehot, axis=0, keepdims=True)

    @pl.when(i == NRB - 1)
    def _finish():
        e_row = lax.broadcasted_iota(jnp.int32, (1, E), 1)
        utri = (lax.broadcasted_iota(jnp.int32, (E, E), 0) <=
                lax.broadcasted_iota(jnp.int32, (E, E), 1)).astype(jnp.float32)
        cnt = run_ref[...]
        end_e = jnp.dot(cnt, utri,
                        preferred_element_type=jnp.float32).astype(jnp.int32)
        cnt_i = cnt.astype(jnp.int32)
        start_e = end_e - cnt_i
        t0 = start_e // TM
        t1 = jnp.where(cnt_i > 0, (end_e - 1) // TM, t0 - 1)
        w_e = jnp.maximum(t1 - t0 + 1, 0)
        cw = jnp.dot(w_e.astype(jnp.float32), utri,
                     preferred_element_type=jnp.float32).astype(jnp.int32)
        cw_excl = cw - w_e
        gid = lax.broadcasted_iota(jnp.int32, (G, 1), 0)
        eg = jnp.sum((cw <= gid).astype(jnp.int32), axis=1,
                     keepdims=True)
        sel = (eg == e_row).astype(jnp.int32)

        def pick(row):
            return jnp.sum(sel * row, axis=1, keepdims=True)

        tile = pick(t0) + (gid - pick(cw_excl))
        total_w = jnp.sum(w_e, axis=1, keepdims=True)
        valid = gid < total_w
        last_e = jnp.max(jnp.where(cnt_i > 0, e_row, 0), axis=1,
                         keepdims=True)
        expert_g = jnp.where(valid, eg, last_e)
        tile_g = jnp.where(valid, tile, NT - 1)
        start_g = jnp.where(valid, pick(start_e), 0)
        end_g = jnp.where(valid, pick(end_e), 0)
        meta_ref[...] = jnp.concatenate(
            [expert_g, tile_g, start_g, end_g], axis=1)
        offs_ref[...] = start_e


def _route(x, Wg, bg):
    return pl.pallas_call(
        _routing_body,
        grid=(NRB,),
        in_specs=[
            pl.BlockSpec((RB, D), lambda i: (i, 0)),
            pl.BlockSpec((D, E), lambda i: (0, 0)),
            pl.BlockSpec((1, E), lambda i: (0, 0)),
        ],
        out_specs=[
            pl.BlockSpec((RB,), lambda i: (i,)),
            pl.BlockSpec((RB,), lambda i: (i,)),
            pl.BlockSpec((1, E), lambda i: (0, 0)),
            pl.BlockSpec((G, 4), lambda i: (0, 0)),
        ],
        out_shape=[
            jax.ShapeDtypeStruct((N,), jnp.int32),
            jax.ShapeDtypeStruct((N,), jnp.int32),
            jax.ShapeDtypeStruct((1, E), jnp.int32),
            jax.ShapeDtypeStruct((G, 4), jnp.int32),
        ],
        scratch_shapes=[pltpu.VMEM((1, E), jnp.float32)],
    )(x, Wg, bg.reshape(1, E))


def _pos_body(ti_ref, rank_ref, offs_ref, pos_ref):
    ti = ti_ref[...]
    sel = (ti[:, None] == lax.broadcasted_iota(jnp.int32, (1, E), 1))
    off = jnp.sum(jnp.where(sel, offs_ref[...], 0), axis=-1)
    pos_ref[...] = off + rank_ref[...]


def _pos(ti, rank, offs):
    return pl.pallas_call(
        _pos_body,
        grid=(NRB,),
        in_specs=[
            pl.BlockSpec((RB,), lambda i: (i,)),
            pl.BlockSpec((RB,), lambda i: (i,)),
            pl.BlockSpec((1, E), lambda i: (0, 0)),
        ],
        out_specs=pl.BlockSpec((RB,), lambda i: (i,)),
        out_shape=jax.ShapeDtypeStruct((N,), jnp.int32),
    )(ti, rank, offs)


def _sc_permute_rows(table, idx, direction):
    info = plsc.get_sparse_core_info()
    nw = info.num_cores * info.num_subcores
    b_per_w = N // nw
    mesh = plsc.VectorSubcoreMesh(core_axis_name="c", subcore_axis_name="s")

    @functools.partial(
        pl.kernel,
        out_type=jax.ShapeDtypeStruct((N, D), jnp.float32),
        mesh=mesh,
        scratch_types=[
            pltpu.VMEM((b_per_w,), jnp.int32),
            pltpu.VMEM((b_per_w, D), jnp.float32),
            pltpu.SemaphoreType.DMA,
        ],
    )
    def permute_kernel(table_hbm, idx_hbm, out_hbm, idx_v, rows_v, sem):
        wid = lax.axis_index("s") * info.num_cores + lax.axis_index("c")
        base = wid * b_per_w
        pltpu.sync_copy(idx_hbm.at[pl.ds(base, b_per_w)], idx_v)
        if direction == "scatter":
            pltpu.sync_copy(table_hbm.at[pl.ds(base, b_per_w)], rows_v)
            pltpu.async_copy(rows_v, out_hbm.at[idx_v], sem).wait()
        else:
            pltpu.async_copy(table_hbm.at[idx_v], rows_v, sem).wait()
            pltpu.sync_copy(rows_v, out_hbm.at[pl.ds(base, b_per_w)])

    return permute_kernel(table, idx)


def _gmm_body(meta_ref, xs_ref, wg0_ref, wg1_ref, wu0_ref, wu1_ref,
              wd0_ref, wd1_ref, out_ref):
    g = pl.program_id(0)
    start = meta_ref[g, 2]
    end = meta_ref[g, 3]

    @pl.when(start < end)
    def _work():
        tile = meta_ref[g, 1]
        xb = xs_ref[...]
        a0 = jnp.dot(xb, wg0_ref[0], preferred_element_type=jnp.float32)
        a1 = jnp.dot(xb, wg1_ref[0], preferred_element_type=jnp.float32)
        b0 = jnp.dot(xb, wu0_ref[0], preferred_element_type=jnp.float32)
        b1 = jnp.dot(xb, wu1_ref[0], preferred_element_type=jnp.float32)
        h0 = a0 * jax.nn.sigmoid(a0) * b0
        h1 = a1 * jax.nn.sigmoid(a1) * b1
        y = (jnp.dot(h0, wd0_ref[0], preferred_element_type=jnp.float32) +
             jnp.dot(h1, wd1_ref[0], preferred_element_type=jnp.float32))
        rows = tile * TM + lax.broadcasted_iota(jnp.int32, (TM, 1), 0)
        mask = (rows >= start) & (rows < end)
        out_ref[...] = jnp.where(mask, y, out_ref[...])


def _gmm(xs, W_gate, W_up, W_down, meta):
    grid_spec = pltpu.PrefetchScalarGridSpec(
        num_scalar_prefetch=1,
        grid=(G,),
        in_specs=[
            pl.BlockSpec((TM, D), lambda g, m: (m[g, 1], 0)),
            pl.BlockSpec((1, D, F // 2), lambda g, m: (m[g, 0], 0, 0)),
            pl.BlockSpec((1, D, F // 2), lambda g, m: (m[g, 0], 0, 1)),
            pl.BlockSpec((1, D, F // 2), lambda g, m: (m[g, 0], 0, 0)),
            pl.BlockSpec((1, D, F // 2), lambda g, m: (m[g, 0], 0, 1)),
            pl.BlockSpec((1, F // 2, D), lambda g, m: (m[g, 0], 0, 0)),
            pl.BlockSpec((1, F // 2, D), lambda g, m: (m[g, 0], 1, 0)),
        ],
        out_specs=pl.BlockSpec((TM, D), lambda g, m: (m[g, 1], 0)),
    )
    return pl.pallas_call(
        _gmm_body,
        grid_spec=grid_spec,
        out_shape=jax.ShapeDtypeStruct((N, D), jnp.float32),
    )(meta, xs, W_gate, W_gate, W_up, W_up, W_down, W_down)


def kernel(x, Wg, bg, W_gate, W_up, W_down):
    top_i, rank, offs, meta = _route(x, Wg, bg)
    pos = _pos(top_i, rank, offs)

    xs = _sc_permute_rows(x, pos, "scatter")
    ys = _gmm(xs, W_gate, W_up, W_down, meta)
    return _sc_permute_rows(ys, pos, "gather")


# speedup vs baseline: 1.0194x; 1.0194x over previous
import functools

import jax
import jax.numpy as jnp
from jax import lax
from jax.experimental import pallas as pl
from jax.experimental.pallas import tpu as pltpu
from jax.experimental.pallas import tpu_sc as plsc

E = 64
D = 1024
F = 1024
N = 2048
TM = 128
NT = N // TM
G = NT + E
RB = 256
NRB = N // RB


def _routing_body(x_ref, wg_ref, bg_ref, pos_ref, meta_ref,
                  run_ref, ti_s, rank_s):
    i = pl.program_id(0)

    @pl.when(i == 0)
    def _init():
        run_ref[...] = jnp.zeros_like(run_ref)

    logits = jnp.dot(x_ref[...], wg_ref[...],
                     preferred_element_type=jnp.float32) + bg_ref[...]
    ti = jnp.argmax(logits, axis=-1).astype(jnp.int32)
    onehot = (ti[:, None] == lax.broadcasted_iota(jnp.int32, (1, E), 1)
              ).astype(jnp.float32)
    r = lax.broadcasted_iota(jnp.int32, (RB, RB), 0)
    c = lax.broadcasted_iota(jnp.int32, (RB, RB), 1)
    ltri = (c < r).astype(jnp.float32)
    prefix = jnp.dot(ltri, onehot, preferred_element_type=jnp.float32)
    totals = run_ref[...]
    rank = jnp.sum((prefix + totals) * onehot, axis=-1)
    ti_s[i] = ti
    rank_s[i] = rank.astype(jnp.int32)
    run_ref[...] = totals + jnp.sum(onehot, axis=0, keepdims=True)

    @pl.when(i == NRB - 1)
    def _finish():
        e_row = lax.broadcasted_iota(jnp.int32, (1, E), 1)
        utri = (lax.broadcasted_iota(jnp.int32, (E, E), 0) <=
                lax.broadcasted_iota(jnp.int32, (E, E), 1)).astype(jnp.float32)
        cnt = run_ref[...]
        end_e = jnp.dot(cnt, utri,
                        preferred_element_type=jnp.float32).astype(jnp.int32)
        cnt_i = cnt.astype(jnp.int32)
        start_e = end_e - cnt_i
        t0 = start_e // TM
        t1 = jnp.where(cnt_i > 0, (end_e - 1) // TM, t0 - 1)
        w_e = jnp.maximum(t1 - t0 + 1, 0)
        cw = jnp.dot(w_e.astype(jnp.float32), utri,
                     preferred_element_type=jnp.float32).astype(jnp.int32)
        cw_excl = cw - w_e
        gid = lax.broadcasted_iota(jnp.int32, (G, 1), 0)
        eg = jnp.sum((cw <= gid).astype(jnp.int32), axis=1,
                     keepdims=True)
        sel = (eg == e_row).astype(jnp.int32)

        def pick(row):
            return jnp.sum(sel * row, axis=1, keepdims=True)

        tile = pick(t0) + (gid - pick(cw_excl))
        total_w = jnp.sum(w_e, axis=1, keepdims=True)
        valid = gid < total_w
        last_e = jnp.max(jnp.where(cnt_i > 0, e_row, 0), axis=1,
                         keepdims=True)
        expert_g = jnp.where(valid, eg, last_e)
        tile_g = jnp.where(valid, tile, NT - 1)
        start_g = jnp.where(valid, pick(start_e), 0)
        end_g = jnp.where(valid, pick(end_e), 0)
        meta_ref[...] = jnp.concatenate(
            [expert_g, tile_g, start_g, end_g], axis=1)
        for j in range(NRB):
            tb = ti_s[j]
            off = jnp.sum(jnp.where(tb[:, None] == e_row, start_e, 0),
                          axis=-1)
            pos_ref[pl.ds(j * RB, RB)] = off + rank_s[j]


def _route(x, Wg, bg):
    return pl.pallas_call(
        _routing_body,
        grid=(NRB,),
        in_specs=[
            pl.BlockSpec((RB, D), lambda i: (i, 0)),
            pl.BlockSpec((D, E), lambda i: (0, 0)),
            pl.BlockSpec((1, E), lambda i: (0, 0)),
        ],
        out_specs=[
            pl.BlockSpec((N,), lambda i: (0,)),
            pl.BlockSpec((G, 4), lambda i: (0, 0)),
        ],
        out_shape=[
            jax.ShapeDtypeStruct((N,), jnp.int32),
            jax.ShapeDtypeStruct((G, 4), jnp.int32),
        ],
        scratch_shapes=[
            pltpu.VMEM((1, E), jnp.float32),
            pltpu.VMEM((NRB, RB), jnp.int32),
            pltpu.VMEM((NRB, RB), jnp.int32),
        ],
    )(x, Wg, bg.reshape(1, E))


def _sc_permute_rows(table, idx, direction):
    info = plsc.get_sparse_core_info()
    nw = info.num_cores * info.num_subcores
    b_per_w = N // nw
    mesh = plsc.VectorSubcoreMesh(core_axis_name="c", subcore_axis_name="s")

    @functools.partial(
        pl.kernel,
        out_type=jax.ShapeDtypeStruct((N, D), jnp.float32),
        mesh=mesh,
        scratch_types=[
            pltpu.VMEM((b_per_w,), jnp.int32),
            pltpu.VMEM((b_per_w, D), jnp.float32),
            pltpu.SemaphoreType.DMA,
        ],
    )
    def permute_kernel(table_hbm, idx_hbm, out_hbm, idx_v, rows_v, sem):
        wid = lax.axis_index("s") * info.num_cores + lax.axis_index("c")
        base = wid * b_per_w
        pltpu.sync_copy(idx_hbm.at[pl.ds(base, b_per_w)], idx_v)
        if direction == "scatter":
            pltpu.sync_copy(table_hbm.at[pl.ds(base, b_per_w)], rows_v)
            pltpu.async_copy(rows_v, out_hbm.at[idx_v], sem).wait()
        else:
            pltpu.async_copy(table_hbm.at[idx_v], rows_v, sem).wait()
            pltpu.sync_copy(rows_v, out_hbm.at[pl.ds(base, b_per_w)])

    return permute_kernel(table, idx)


def _gmm_body(meta_ref, xs_ref, wg_ref, wu_ref, wd_ref, out_ref):
    g = pl.program_id(0)
    start = meta_ref[g, 2]
    end = meta_ref[g, 3]

    @pl.when(start < end)
    def _work():
        tile = meta_ref[g, 1]
        xb = xs_ref[...]
        a = jnp.dot(xb, wg_ref[0], preferred_element_type=jnp.float32)
        b = jnp.dot(xb, wu_ref[0], preferred_element_type=jnp.float32)
        h = a * jax.nn.sigmoid(a) * b
        y = jnp.dot(h, wd_ref[0], preferred_element_type=jnp.float32)
        rows = tile * TM + lax.broadcasted_iota(jnp.int32, (TM, 1), 0)
        mask = (rows >= start) & (rows < end)
        out_ref[...] = jnp.where(mask, y, out_ref[...])


def _gmm(xs, W_gate, W_up, W_down, meta):
    grid_spec = pltpu.PrefetchScalarGridSpec(
        num_scalar_prefetch=1,
        grid=(G,),
        in_specs=[
            pl.BlockSpec((TM, D), lambda g, m: (m[g, 1], 0)),
            pl.BlockSpec((1, D, F), lambda g, m: (m[g, 0], 0, 0)),
            pl.BlockSpec((1, D, F), lambda g, m: (m[g, 0], 0, 0)),
            pl.BlockSpec((1, F, D), lambda g, m: (m[g, 0], 0, 0)),
        ],
        out_specs=pl.BlockSpec((TM, D), lambda g, m: (m[g, 1], 0)),
    )
    return pl.pallas_call(
        _gmm_body,
        grid_spec=grid_spec,
        out_shape=jax.ShapeDtypeStruct((N, D), jnp.float32),
    )(meta, xs, W_gate, W_up, W_down)


def kernel(x, Wg, bg, W_gate, W_up, W_down):
    pos, meta = _route(x, Wg, bg)

    xs = _sc_permute_rows(x, pos, "scatter")
    ys = _gmm(xs, W_gate, W_up, W_down, meta)
    return _sc_permute_rows(ys, pos, "gather")
